# baseline (speedup 1.0000x reference)
import functools

import jax
import jax.numpy as jnp
from jax import lax
from jax.experimental import pallas as pl
from jax.experimental.pallas import tpu as pltpu
from jax.experimental.pallas import tpu_sc as plsc

_NUM_CORES = 2
_NUM_SUBCORES = 16
_NUM_WORKERS = _NUM_CORES * _NUM_SUBCORES

_CHUNK = 80
_GCHUNKS = 5
_GROUP = _CHUNK * _GCHUNKS


def _sc_gather(features, idx_flat):
    e_total = idx_flat.shape[0]
    d = features.shape[1]
    dt = features.dtype
    per_w = e_total // _NUM_WORKERS
    n_chunks = per_w // _CHUNK
    n_groups = per_w // _GROUP
    assert per_w * _NUM_WORKERS == e_total
    assert n_chunks * _CHUNK == per_w and n_groups * _GROUP == per_w
    assert _GROUP % 8 == 0 and _CHUNK % 8 == 0
    idx3 = idx_flat.reshape(_NUM_WORKERS, n_chunks, _CHUNK)

    mesh = plsc.VectorSubcoreMesh(core_axis_name="c", subcore_axis_name="s")

    @functools.partial(
        pl.kernel,
        out_type=jax.ShapeDtypeStruct((e_total, d), dt),
        mesh=mesh,
        compiler_params=pltpu.CompilerParams(use_tc_tiling_on_sc=False),
        scratch_types=[
            pltpu.VMEM((n_chunks, _CHUNK), jnp.int32),
            pltpu.VMEM((_GROUP, d), dt),
            pltpu.VMEM((_GROUP, d), dt),
            pltpu.SemaphoreType.DMA,
            pltpu.SemaphoreType.DMA,
            pltpu.SemaphoreType.DMA,
        ],
    )
    def gather_kernel(table_hbm, idx_hbm, out_hbm,
                      idx_v, buf0, buf1, sem_g, sem_w0, sem_w1):
        wid = lax.axis_index("s") * _NUM_CORES + lax.axis_index("c")
        base = wid * per_w
        pltpu.sync_copy(idx_hbm.at[wid], idx_v)

        def run_group(g, buf, sem_w, first):
            wb = pltpu.make_async_copy(
                buf, out_hbm.at[pl.ds(base, _GROUP)], sem_w)
            pl.when(jnp.logical_not(first))(wb.wait)
            cps = []
            for i in range(_GCHUNKS):
                cp = pltpu.make_async_copy(
                    table_hbm.at[idx_v.at[g * _GCHUNKS + i]],
                    buf.at[pl.ds(i * _CHUNK, _CHUNK)], sem_g)
                cp.start()
                cps.append(cp)
            for cp in cps:
                cp.wait()
            pltpu.make_async_copy(
                buf, out_hbm.at[pl.ds(base + g * _GROUP, _GROUP)], sem_w).start()

        def body(t, _):
            run_group(2 * t, buf0, sem_w0, t == 0)
            run_group(2 * t + 1, buf1, sem_w1, t == 0)
            return 0

        n_pairs = n_groups // 2
        lax.fori_loop(0, n_pairs, body, 0, unroll=False)
        if n_groups % 2:
            run_group(n_groups - 1, buf0, sem_w0, jnp.bool_(n_pairs == 0))
        pltpu.make_async_copy(
            buf0, out_hbm.at[pl.ds(base, _GROUP)], sem_w0).wait()
        pltpu.make_async_copy(
            buf1, out_hbm.at[pl.ds(base, _GROUP)], sem_w1).wait()

    return gather_kernel(features, idx3)


def _tc_compute(features, nf, w1t, b1, w2, block_n):
    n, d = features.shape
    k = nf.shape[0] // n
    assert n % block_n == 0

    def body(f_ref, nf_ref, w1t_ref, b1_ref, w2_ref, out_ref):
        f = f_ref[...].astype(jnp.bfloat16)
        w1t_full = w1t_ref[...]
        a = jnp.dot(f, w1t_full[:d, :], preferred_element_type=jnp.float32)
        a = a + b1_ref[...]
        nfb = nf_ref[...]
        t = jnp.dot(nfb, w1t_full[d:, :], preferred_element_type=jnp.float32)
        h = jnp.maximum(t.reshape(block_n, k, d) + a[:, None, :], 0.0)
        s = jnp.sum(h * w2_ref[...][None, :, :], axis=-1)
        m = jnp.max(s, axis=-1, keepdims=True)
        e = jnp.exp(s - m)
        w = e / jnp.sum(e, axis=-1, keepdims=True)
        out_ref[...] = jnp.sum(
            nfb.reshape(block_n, k, d).astype(jnp.float32) * w[:, :, None],
            axis=1)

    return pl.pallas_call(
        body,
        grid=(n // block_n,),
        in_specs=[
            pl.BlockSpec((block_n, d), lambda i: (i, 0)),
            pl.BlockSpec((block_n * k, d), lambda i: (i, 0)),
            pl.BlockSpec((2 * d, d), lambda i: (0, 0)),
            pl.BlockSpec((1, d), lambda i: (0, 0)),
            pl.BlockSpec((1, d), lambda i: (0, 0)),
        ],
        out_specs=pl.BlockSpec((block_n, d), lambda i: (i, 0)),
        out_shape=jax.ShapeDtypeStruct((n, d), jnp.float32),
    )(features, nf, w1t, b1, w2)


def kernel(features, neighbors, W1, b1, W2, b2):
    n, d = features.shape
    idx_flat = neighbors.reshape(-1).astype(jnp.int32)
    feat_bf16 = features.astype(jnp.bfloat16)
    tab_i32 = jax.lax.bitcast_convert_type(
        feat_bf16.reshape(n, d // 2, 2), jnp.int32)
    nf_i32 = _sc_gather(tab_i32, idx_flat)
    nf_bf16 = jax.lax.bitcast_convert_type(
        nf_i32, jnp.bfloat16).reshape(idx_flat.shape[0], d)
    w1t = W1.T.reshape(2 * d, d).astype(jnp.bfloat16)
    b1r = b1.reshape(1, d)
    w2r = W2.reshape(1, d)
    return _tc_compute(features, nf_bf16, w1t, b1r, w2r, block_n=200)


# speedup vs baseline: 2.8270x; 2.8270x over previous
import functools

import jax
import jax.numpy as jnp
from jax import lax
from jax.experimental import pallas as pl
from jax.experimental.pallas import tpu as pltpu
from jax.experimental.pallas import tpu_sc as plsc

_NUM_CORES = 2
_NUM_SUBCORES = 16
_NUM_WORKERS = _NUM_CORES * _NUM_SUBCORES

_CHUNK = 80
_GCHUNKS = 5
_GROUP = _CHUNK * _GCHUNKS


def _sc_gather(features, idx_flat):
    e_total = idx_flat.shape[0]
    d = features.shape[1]
    dt = features.dtype
    per_w = e_total // _NUM_WORKERS
    n_chunks = per_w // _CHUNK
    n_groups = per_w // _GROUP
    assert per_w * _NUM_WORKERS == e_total
    assert n_chunks * _CHUNK == per_w and n_groups * _GROUP == per_w
    assert _GROUP % 8 == 0 and _CHUNK % 8 == 0
    idx3 = idx_flat.reshape(_NUM_WORKERS, n_chunks, _CHUNK)

    mesh = plsc.VectorSubcoreMesh(core_axis_name="c", subcore_axis_name="s")

    @functools.partial(
        pl.kernel,
        out_type=jax.ShapeDtypeStruct((e_total, d), dt),
        mesh=mesh,
        compiler_params=pltpu.CompilerParams(use_tc_tiling_on_sc=False),
        scratch_types=[
            pltpu.VMEM((n_chunks, _CHUNK), jnp.int32),
            pltpu.VMEM((_GROUP, d), dt),
            pltpu.VMEM((_GROUP, d), dt),
            pltpu.SemaphoreType.DMA,
            pltpu.SemaphoreType.DMA,
            pltpu.SemaphoreType.DMA,
        ],
    )
    def gather_kernel(table_hbm, idx_hbm, out_hbm,
                      idx_v, buf0, buf1, sem_g, sem_w0, sem_w1):
        wid = lax.axis_index("s") * _NUM_CORES + lax.axis_index("c")
        base = wid * per_w
        pltpu.sync_copy(idx_hbm.at[wid], idx_v)

        def run_group(g, buf, sem_w, first):
            wb = pltpu.make_async_copy(
                buf, out_hbm.at[pl.ds(base, _GROUP)], sem_w)
            pl.when(jnp.logical_not(first))(wb.wait)
            cps = []
            for i in range(_GCHUNKS):
                cp = pltpu.make_async_copy(
                    table_hbm.at[idx_v.at[g * _GCHUNKS + i]],
                    buf.at[pl.ds(i * _CHUNK, _CHUNK)], sem_g)
                cp.start()
                cps.append(cp)
            for cp in cps:
                cp.wait()
            pltpu.make_async_copy(
                buf, out_hbm.at[pl.ds(base + g * _GROUP, _GROUP)], sem_w).start()

        def body(t, _):
            run_group(2 * t, buf0, sem_w0, t == 0)
            run_group(2 * t + 1, buf1, sem_w1, t == 0)
            return 0

        n_pairs = n_groups // 2
        lax.fori_loop(0, n_pairs, body, 0, unroll=False)
        if n_groups % 2:
            run_group(n_groups - 1, buf0, sem_w0, jnp.bool_(n_pairs == 0))
        pltpu.make_async_copy(
            buf0, out_hbm.at[pl.ds(base, _GROUP)], sem_w0).wait()
        pltpu.make_async_copy(
            buf1, out_hbm.at[pl.ds(base, _GROUP)], sem_w1).wait()

    return gather_kernel(features, idx3)


def _tc_compute(features, nf, w1t, b1, w2, block_n):
    n, d = features.shape
    k = nf.shape[0] // n
    assert n % block_n == 0

    def body(f_ref, nf_ref, w1t_ref, b1_ref, w2_ref, out_ref):
        f = f_ref[...].astype(jnp.bfloat16)
        w1t_full = w1t_ref[...]
        a = jnp.dot(f, w1t_full[:d, :], preferred_element_type=jnp.float32)
        a = a + b1_ref[...]
        wrd = nf_ref[...]
        lo = jax.lax.bitcast_convert_type(wrd << 16, jnp.float32)
        hi = jax.lax.bitcast_convert_type(
            wrd & jnp.int32(-65536), jnp.float32)
        nff = jnp.concatenate([lo, hi], axis=-1)
        nfb = nff.astype(jnp.bfloat16)
        t = jnp.dot(nfb, w1t_full[d:, :], preferred_element_type=jnp.float32)
        h = jnp.maximum(t.reshape(block_n, k, d) + a[:, None, :], 0.0)
        s = jnp.sum(h * w2_ref[...][None, :, :], axis=-1)
        m = jnp.max(s, axis=-1, keepdims=True)
        e = jnp.exp(s - m)
        w = e / jnp.sum(e, axis=-1, keepdims=True)
        out_ref[...] = jnp.sum(
            nff.reshape(block_n, k, d) * w[:, :, None], axis=1)

    return pl.pallas_call(
        body,
        grid=(n // block_n,),
        in_specs=[
            pl.BlockSpec((block_n, d), lambda i: (i, 0)),
            pl.BlockSpec((block_n * k, d // 2), lambda i: (i, 0)),
            pl.BlockSpec((2 * d, d), lambda i: (0, 0)),
            pl.BlockSpec((1, d), lambda i: (0, 0)),
            pl.BlockSpec((1, d), lambda i: (0, 0)),
        ],
        out_specs=pl.BlockSpec((block_n, d), lambda i: (i, 0)),
        out_shape=jax.ShapeDtypeStruct((n, d), jnp.float32),
    )(features, nf, w1t, b1, w2)


def kernel(features, neighbors, W1, b1, W2, b2):
    n, d = features.shape
    idx_flat = neighbors.reshape(-1).astype(jnp.int32)
    feat_bf16 = features.astype(jnp.bfloat16)
    lo_u = jax.lax.bitcast_convert_type(
        feat_bf16[:, :d // 2], jnp.uint16).astype(jnp.uint32)
    hi_u = jax.lax.bitcast_convert_type(
        feat_bf16[:, d // 2:], jnp.uint16).astype(jnp.uint32)
    tab_i32 = jax.lax.bitcast_convert_type(
        lo_u | (hi_u << 16), jnp.int32)
    nf_i32 = _sc_gather(tab_i32, idx_flat)
    w1t = W1.T.reshape(2 * d, d).astype(jnp.bfloat16)
    b1r = b1.reshape(1, d)
    w2r = W2.reshape(1, d)
    return _tc_compute(features, nf_i32, w1t, b1r, w2r, block_n=200)


# speedup vs baseline: 4.2427x; 1.5008x over previous
import functools

import jax
import jax.numpy as jnp
from jax import lax
from jax.experimental import pallas as pl
from jax.experimental.pallas import tpu as pltpu
from jax.experimental.pallas import tpu_sc as plsc

_NUM_CORES = 2
_NUM_SUBCORES = 16
_NUM_WORKERS = _NUM_CORES * _NUM_SUBCORES

_CHUNK = 80
_GCHUNKS = 5
_GROUP = _CHUNK * _GCHUNKS

_SLICES = 5
_BLOCK_N = 200


def _sc_gather(features, idx_flat):
    e_total = idx_flat.shape[0]
    d = features.shape[1]
    dt = features.dtype
    per_w = e_total // _NUM_WORKERS
    n_chunks = per_w // _CHUNK
    n_groups = per_w // _GROUP
    assert per_w * _NUM_WORKERS == e_total
    assert n_chunks * _CHUNK == per_w and n_groups * _GROUP == per_w
    assert per_w % 8 == 0 and _CHUNK % 8 == 0
    idx3 = idx_flat.reshape(_NUM_WORKERS, n_chunks, _CHUNK)

    mesh = plsc.VectorSubcoreMesh(core_axis_name="c", subcore_axis_name="s")

    @functools.partial(
        pl.kernel,
        out_type=jax.ShapeDtypeStruct((e_total, d), dt),
        mesh=mesh,
        scratch_types=[
            pltpu.VMEM((n_chunks, _CHUNK), jnp.int32),
            pltpu.VMEM((_GROUP, d), dt),
            pltpu.VMEM((_GROUP, d), dt),
            pltpu.SemaphoreType.DMA,
            pltpu.SemaphoreType.DMA,
            pltpu.SemaphoreType.DMA,
        ],
    )
    def gather_kernel(table_hbm, idx_hbm, out_hbm,
                      idx_v, buf0, buf1, sem_g, sem_w0, sem_w1):
        wid = lax.axis_index("s") * _NUM_CORES + lax.axis_index("c")
        base = wid * per_w
        pltpu.sync_copy(idx_hbm.at[wid], idx_v)

        def run_group(g, buf, sem_w, first):
            wb = pltpu.make_async_copy(
                buf, out_hbm.at[pl.ds(base, _GROUP)], sem_w)
            pl.when(jnp.logical_not(first))(wb.wait)
            cps = []
            for i in range(_GCHUNKS):
                cp = pltpu.make_async_copy(
                    table_hbm.at[idx_v.at[g * _GCHUNKS + i]],
                    buf.at[pl.ds(i * _CHUNK, _CHUNK)], sem_g)
                cp.start()
                cps.append(cp)
            for cp in cps:
                cp.wait()
            pltpu.make_async_copy(
                buf, out_hbm.at[pl.ds(base + g * _GROUP, _GROUP)], sem_w).start()

        def body(t, _):
            run_group(2 * t, buf0, sem_w0, t == 0)
            run_group(2 * t + 1, buf1, sem_w1, t == 0)
            return 0

        n_pairs = n_groups // 2
        lax.fori_loop(0, n_pairs, body, 0, unroll=False)
        if n_groups % 2:
            run_group(n_groups - 1, buf0, sem_w0, jnp.bool_(n_pairs == 0))
        pltpu.make_async_copy(
            buf0, out_hbm.at[pl.ds(base, _GROUP)], sem_w0).wait()
        pltpu.make_async_copy(
            buf1, out_hbm.at[pl.ds(base, _GROUP)], sem_w1).wait()

    return gather_kernel(features, idx3)


def _tc_compute(features, nf, w1t, b1, w2, block_n):
    n, d = features.shape
    k = nf.shape[0] // n
    assert n % block_n == 0

    def body(f_ref, nf_ref, w1t_ref, b1_ref, w2_ref, out_ref):
        f = f_ref[...].astype(jnp.bfloat16)
        w1t_full = w1t_ref[...]
        a = jnp.dot(f, w1t_full[:d, :], preferred_element_type=jnp.float32)
        a = a + b1_ref[...]
        nff = nf_ref[...]
        nfb = nff.astype(jnp.bfloat16)
        t = jnp.dot(nfb, w1t_full[d:, :], preferred_element_type=jnp.float32)
        h = jnp.maximum(t.reshape(block_n, k, d) + a[:, None, :], 0.0)
        s = jnp.sum(h * w2_ref[...][None, :, :], axis=-1)
        m = jnp.max(s, axis=-1, keepdims=True)
        e = jnp.exp(s - m)
        w = e / jnp.sum(e, axis=-1, keepdims=True)
        out_ref[...] = jnp.sum(
            nff.reshape(block_n, k, d) * w[:, :, None], axis=1)

    return pl.pallas_call(
        body,
        grid=(n // block_n,),
        in_specs=[
            pl.BlockSpec((block_n, d), lambda i: (i, 0)),
            pl.BlockSpec((block_n * k, d), lambda i: (i, 0)),
            pl.BlockSpec((2 * d, d), lambda i: (0, 0)),
            pl.BlockSpec((1, d), lambda i: (0, 0)),
            pl.BlockSpec((1, d), lambda i: (0, 0)),
        ],
        out_specs=pl.BlockSpec((block_n, d), lambda i: (i, 0)),
        out_shape=jax.ShapeDtypeStruct((n, d), jnp.float32),
    )(features, nf, w1t, b1, w2)


def kernel(features, neighbors, W1, b1, W2, b2):
    n, d = features.shape
    k_n = neighbors.shape[1]
    idx_flat = neighbors.reshape(-1).astype(jnp.int32)
    w1t = W1.T.reshape(2 * d, d).astype(jnp.bfloat16)
    b1r = b1.reshape(1, d)
    w2r = W2.reshape(1, d)
    ns = n // _SLICES
    es = ns * k_n
    outs = []
    for s in range(_SLICES):
        nf_s = _sc_gather(features, idx_flat[s * es:(s + 1) * es])
        outs.append(_tc_compute(
            features[s * ns:(s + 1) * ns], nf_s, w1t, b1r, w2r, _BLOCK_N))
    return jnp.concatenate(outs, axis=0)
